# Initial kernel scaffold; baseline (speedup 1.0000x reference)
import jax
import jax.numpy as jnp
from jax.experimental import pallas as pl


def kernel(x, edge_index, edge_weight, W1, b1, W2, b2):
    raise NotImplementedError("write your pallas kernel here")


# speedup vs baseline: 2.6607x; 2.6607x over previous
import jax
import jax.numpy as jnp
from jax import lax
from jax.experimental import pallas as pl
from jax.experimental.pallas import tpu as pltpu
from jax.experimental.pallas import tpu_sc as plsc

N_NODES = 10000
D = 128
N_EDGES = 320000

NC = 2
NS = 16
NTILES = NC * NS
CHUNK = 128
E_PER_TILE = 10240
E_PAD = E_PER_TILE * NTILES
NCHUNKS = E_PER_TILE // CHUNK
N_PAD = 10240
ROWS_PER_SUB = N_PAD // NS
ZCH = 32

_mesh = plsc.VectorSubcoreMesh(core_axis_name="c", subcore_axis_name="s")


def _spmm_body(sup, srcr, dstr, wr, out, acc, src_v, dst_v, w_v, rows_v,
               stage_v, gsem):
    cid = lax.axis_index("c")
    sid = lax.axis_index("s")
    wid = sid * NC + cid

    for i in range(ZCH):
        for j in range(8):
            stage_v[i, pl.ds(j * 16, 16)] = jnp.zeros((16,), jnp.float32)
    row0 = sid * ROWS_PER_SUB

    def zloop(i, carry):
        pltpu.sync_copy(stage_v, acc.at[pl.ds(row0 + i * ZCH, ZCH)])
        return carry

    lax.fori_loop(0, ROWS_PER_SUB // ZCH, zloop, 0)
    plsc.subcore_barrier()

    ebase = wid * E_PER_TILE

    def chunk_body(g, carry):
        off = ebase + g * CHUNK
        pltpu.sync_copy(srcr.at[pl.ds(off, CHUNK)], src_v)
        pltpu.sync_copy(dstr.at[pl.ds(off, CHUNK)], dst_v)
        pltpu.sync_copy(wr.at[pl.ds(off, CHUNK)], w_v)
        pltpu.async_copy(sup.at[src_v], rows_v, gsem).wait()

        def scale_block(b, c2):
            wv = w_v[pl.ds(b * 16, 16)]
            e0 = b * 16
            for l in range(16):
                ws = wv[l]
                for j in range(8):
                    sl = pl.ds(j * 16, 16)
                    rows_v[e0 + l, sl] = rows_v[e0 + l, sl] * ws
            return c2

        lax.fori_loop(0, CHUNK // 16, scale_block, 0)
        pltpu.sync_copy(rows_v, acc.at[dst_v], add=True)
        return carry

    lax.fori_loop(0, NCHUNKS, chunk_body, 0)
    plsc.subcore_barrier()

    def oloop(i, carry):
        r0 = row0 + i * ZCH
        pltpu.sync_copy(acc.at[pl.ds(r0, ZCH)], stage_v)
        pltpu.sync_copy(stage_v, out.at[cid, pl.ds(r0, ZCH)])
        return carry

    lax.fori_loop(0, ROWS_PER_SUB // ZCH, oloop, 0)


_spmm = pl.kernel(
    _spmm_body,
    out_type=jax.ShapeDtypeStruct((NC, N_PAD, D), jnp.float32),
    mesh=_mesh,
    scratch_types=[
        pltpu.VMEM_SHARED((N_PAD, D), jnp.float32),
        pltpu.VMEM((CHUNK,), jnp.int32),
        pltpu.VMEM((CHUNK,), jnp.int32),
        pltpu.VMEM((CHUNK,), jnp.float32),
        pltpu.VMEM((CHUNK, D), jnp.float32),
        pltpu.VMEM((ZCH, D), jnp.float32),
        pltpu.SemaphoreType.DMA,
    ],
)

_MBLK = 2000


def _mm1_body(x_ref, w_ref, b_ref, o_ref):
    o_ref[...] = (
        jnp.dot(x_ref[...], w_ref[...], preferred_element_type=jnp.float32)
        + b_ref[...]
    )


def _dense1(x, wt, b):
    return pl.pallas_call(
        _mm1_body,
        grid=(N_NODES // _MBLK,),
        in_specs=[
            pl.BlockSpec((_MBLK, D), lambda i: (i, 0)),
            pl.BlockSpec((D, D), lambda i: (0, 0)),
            pl.BlockSpec((1, D), lambda i: (0, 0)),
        ],
        out_specs=pl.BlockSpec((_MBLK, D), lambda i: (i, 0)),
        out_shape=jax.ShapeDtypeStruct((N_NODES, D), jnp.float32),
    )(x, wt, b.reshape(1, D))


def _mm2_body(hp_ref, w_ref, b_ref, o_ref):
    h = jax.nn.relu(hp_ref[0] + hp_ref[1])
    o_ref[...] = (
        jnp.dot(h, w_ref[...], preferred_element_type=jnp.float32)
        + b_ref[...]
    )


def _dense2(hp, wt, b):
    return pl.pallas_call(
        _mm2_body,
        grid=(N_NODES // _MBLK,),
        in_specs=[
            pl.BlockSpec((NC, _MBLK, D), lambda i: (0, i, 0)),
            pl.BlockSpec((D, D), lambda i: (0, 0)),
            pl.BlockSpec((1, D), lambda i: (0, 0)),
        ],
        out_specs=pl.BlockSpec((_MBLK, D), lambda i: (i, 0)),
        out_shape=jax.ShapeDtypeStruct((N_NODES, D), jnp.float32),
    )(hp, wt, b.reshape(1, D))


def _add_body(p_ref, o_ref):
    o_ref[...] = p_ref[0] + p_ref[1]


def _combine(p):
    return pl.pallas_call(
        _add_body,
        grid=(N_NODES // _MBLK,),
        in_specs=[pl.BlockSpec((NC, _MBLK, D), lambda i: (0, i, 0))],
        out_specs=pl.BlockSpec((_MBLK, D), lambda i: (i, 0)),
        out_shape=jax.ShapeDtypeStruct((N_NODES, D), jnp.float32),
    )(p)


def kernel(x, edge_index, edge_weight, W1, b1, W2, b2):
    src = edge_index[1].astype(jnp.int32)
    dst = edge_index[0].astype(jnp.int32)
    npad = E_PAD - src.shape[0]
    src_p = jnp.concatenate([src, jnp.zeros((npad,), jnp.int32)])
    dst_p = jnp.concatenate([dst, jnp.zeros((npad,), jnp.int32)])
    w_p = jnp.concatenate(
        [edge_weight.astype(jnp.float32), jnp.zeros((npad,), jnp.float32)]
    )

    s1 = _dense1(x, W1.T, b1)
    hp = _spmm(s1, src_p, dst_p, w_p)
    s2 = _dense2(hp, W2.T, b2)
    op = _spmm(s2, src_p, dst_p, w_p)
    return _combine(op)


# speedup vs baseline: 3.3612x; 1.2633x over previous
import jax
import jax.numpy as jnp
from jax import lax
from jax.experimental import pallas as pl
from jax.experimental.pallas import tpu as pltpu
from jax.experimental.pallas import tpu_sc as plsc

N_NODES = 10000
D = 128
N_EDGES = 320000

NC = 2
NS = 16
NTILES = NC * NS
CHUNK = 128
E_PER_TILE = 10240
E_PAD = E_PER_TILE * NTILES
NCHUNKS = E_PER_TILE // CHUNK
N_PAD = 10240
ROWS_PER_SUB = N_PAD // NS
ZCH = 32

_mesh = plsc.VectorSubcoreMesh(core_axis_name="c", subcore_axis_name="s")


GRP = 8
NBUF = 2


def _spmm_body(sup, srcr, dstr, wr, out, acc, srcg, dstg, wg, rows_v,
               stage_v, gsem, ssem):
    cid = lax.axis_index("c")
    sid = lax.axis_index("s")
    wid = sid * NC + cid

    for i in range(ZCH):
        for j in range(8):
            stage_v[i, pl.ds(j * 16, 16)] = jnp.zeros((16,), jnp.float32)
    row0 = sid * ROWS_PER_SUB

    def zloop(i, carry):
        pltpu.sync_copy(stage_v, acc.at[pl.ds(row0 + i * ZCH, ZCH)])
        return carry

    lax.fori_loop(0, ROWS_PER_SUB // ZCH, zloop, 0)
    plsc.subcore_barrier()

    gbase = wid * NCHUNKS

    def _load_group(c):
        p = lax.rem(lax.div(c, GRP), 2)
        row = pl.multiple_of(gbase + c, 8)
        pltpu.sync_copy(srcr.at[pl.ds(row, GRP)], srcg.at[p])
        pltpu.sync_copy(dstr.at[pl.ds(row, GRP)], dstg.at[p])
        pltpu.sync_copy(wr.at[pl.ds(row, GRP)], wg.at[p])

    def _issue_gather(c):
        p = lax.rem(lax.div(c, GRP), 2)
        g = lax.rem(c, GRP)
        slot = lax.rem(c, NBUF)
        pltpu.async_copy(sup.at[srcg.at[p, g]], rows_v.at[slot], gsem)

    def _wait_one(sem):
        pltpu.make_async_copy(sup.at[pl.ds(0, CHUNK)], rows_v.at[0], sem).wait()

    def _scale_and_fire(c):
        p = lax.rem(lax.div(c, GRP), 2)
        g = lax.rem(c, GRP)
        slot = lax.rem(c, NBUF)

        def scale_block(b, c2):
            wv = wg[p, g, pl.ds(b * 16, 16)]
            e0 = b * 16
            for l in range(16):
                ws = wv[l]
                for j in range(8):
                    sl = pl.ds(j * 16, 16)
                    rows_v[slot, e0 + l, sl] = rows_v[slot, e0 + l, sl] * ws
            return c2

        lax.fori_loop(0, CHUNK // 16, scale_block, 0)
        pltpu.async_copy(rows_v.at[slot], acc.at[dstg.at[p, g]], ssem,
                         add=True)

    def chunk_body(c, carry):
        @pl.when(lax.rem(c, GRP) == 0)
        def _():
            _load_group(c)

        @pl.when(c >= NBUF)
        def _():
            _wait_one(ssem)

        _issue_gather(c)

        @pl.when(c >= 1)
        def _():
            _wait_one(gsem)
            _scale_and_fire(c - 1)

        return carry

    lax.fori_loop(0, NCHUNKS, chunk_body, 0)
    _wait_one(gsem)
    _scale_and_fire(NCHUNKS - 1)
    for _ in range(NBUF):
        _wait_one(ssem)
    plsc.subcore_barrier()

    def oloop(i, carry):
        r0 = row0 + i * ZCH
        pltpu.sync_copy(acc.at[pl.ds(r0, ZCH)], stage_v)
        pltpu.sync_copy(stage_v, out.at[cid, pl.ds(r0, ZCH)])
        return carry

    lax.fori_loop(0, ROWS_PER_SUB // ZCH, oloop, 0)


_spmm = pl.kernel(
    _spmm_body,
    out_type=jax.ShapeDtypeStruct((NC, N_PAD, D), jnp.float32),
    mesh=_mesh,
    scratch_types=[
        pltpu.VMEM_SHARED((N_PAD, D), jnp.float32),
        pltpu.VMEM((2, GRP, CHUNK), jnp.int32),
        pltpu.VMEM((2, GRP, CHUNK), jnp.int32),
        pltpu.VMEM((2, GRP, CHUNK), jnp.float32),
        pltpu.VMEM((NBUF, CHUNK, D), jnp.float32),
        pltpu.VMEM((ZCH, D), jnp.float32),
        pltpu.SemaphoreType.DMA,
        pltpu.SemaphoreType.DMA,
    ],
)

_MBLK = 2000


def _mm1_body(x_ref, w_ref, b_ref, o_ref):
    o_ref[...] = (
        jnp.dot(x_ref[...], w_ref[...], preferred_element_type=jnp.float32)
        + b_ref[...]
    )


def _dense1(x, wt, b):
    return pl.pallas_call(
        _mm1_body,
        grid=(N_NODES // _MBLK,),
        in_specs=[
            pl.BlockSpec((_MBLK, D), lambda i: (i, 0)),
            pl.BlockSpec((D, D), lambda i: (0, 0)),
            pl.BlockSpec((1, D), lambda i: (0, 0)),
        ],
        out_specs=pl.BlockSpec((_MBLK, D), lambda i: (i, 0)),
        out_shape=jax.ShapeDtypeStruct((N_NODES, D), jnp.float32),
    )(x, wt, b.reshape(1, D))


def _mm2_body(hp_ref, w_ref, b_ref, o_ref):
    h = jax.nn.relu(hp_ref[0] + hp_ref[1])
    o_ref[...] = (
        jnp.dot(h, w_ref[...], preferred_element_type=jnp.float32)
        + b_ref[...]
    )


def _dense2(hp, wt, b):
    return pl.pallas_call(
        _mm2_body,
        grid=(N_NODES // _MBLK,),
        in_specs=[
            pl.BlockSpec((NC, _MBLK, D), lambda i: (0, i, 0)),
            pl.BlockSpec((D, D), lambda i: (0, 0)),
            pl.BlockSpec((1, D), lambda i: (0, 0)),
        ],
        out_specs=pl.BlockSpec((_MBLK, D), lambda i: (i, 0)),
        out_shape=jax.ShapeDtypeStruct((N_NODES, D), jnp.float32),
    )(hp, wt, b.reshape(1, D))


def _add_body(p_ref, o_ref):
    o_ref[...] = p_ref[0] + p_ref[1]


def _combine(p):
    return pl.pallas_call(
        _add_body,
        grid=(N_NODES // _MBLK,),
        in_specs=[pl.BlockSpec((NC, _MBLK, D), lambda i: (0, i, 0))],
        out_specs=pl.BlockSpec((_MBLK, D), lambda i: (i, 0)),
        out_shape=jax.ShapeDtypeStruct((N_NODES, D), jnp.float32),
    )(p)


def kernel(x, edge_index, edge_weight, W1, b1, W2, b2):
    src = edge_index[1].astype(jnp.int32)
    dst = edge_index[0].astype(jnp.int32)
    npad = E_PAD - src.shape[0]
    src_p = jnp.concatenate([src, jnp.zeros((npad,), jnp.int32)])
    dst_p = jnp.concatenate([dst, jnp.zeros((npad,), jnp.int32)])
    w_p = jnp.concatenate(
        [edge_weight.astype(jnp.float32), jnp.zeros((npad,), jnp.float32)]
    )
    src_p = src_p.reshape(E_PAD // CHUNK, CHUNK)
    dst_p = dst_p.reshape(E_PAD // CHUNK, CHUNK)
    w_p = w_p.reshape(E_PAD // CHUNK, CHUNK)

    s1 = _dense1(x, W1.T, b1)
    hp = _spmm(s1, src_p, dst_p, w_p)
    s2 = _dense2(hp, W2.T, b2)
    op = _spmm(s2, src_p, dst_p, w_p)
    return _combine(op)


# speedup vs baseline: 3.4446x; 1.0248x over previous
import jax
import jax.numpy as jnp
from jax import lax
from jax.experimental import pallas as pl
from jax.experimental.pallas import tpu as pltpu
from jax.experimental.pallas import tpu_sc as plsc

N_NODES = 10000
D = 128
N_EDGES = 320000

NC = 2
NS = 16
NTILES = NC * NS
CHUNK = 128
E_PER_TILE = 10240
E_PAD = E_PER_TILE * NTILES
NCHUNKS = E_PER_TILE // CHUNK
N_PAD = 10240
ROWS_PER_SUB = N_PAD // NS
ZCH = 32

_mesh = plsc.VectorSubcoreMesh(core_axis_name="c", subcore_axis_name="s")


GRP = 8
NBUF = 2


def _spmm_body(sup, srcr, dstr, wr, out, acc, srcg, dstg, wg, rows_v,
               stage_v, gsem, ssem, isem):
    cid = lax.axis_index("c")
    sid = lax.axis_index("s")
    wid = sid * NC + cid

    for i in range(ZCH):
        for j in range(8):
            stage_v[i, pl.ds(j * 16, 16)] = jnp.zeros((16,), jnp.float32)
    row0 = sid * ROWS_PER_SUB

    def zloop(i, carry):
        pltpu.sync_copy(stage_v, acc.at[pl.ds(row0 + i * ZCH, ZCH)])
        return carry

    lax.fori_loop(0, ROWS_PER_SUB // ZCH, zloop, 0)
    plsc.subcore_barrier()

    gbase = wid * NCHUNKS
    NGRP = NCHUNKS // GRP

    def _load_group_sync(go, p):
        row = pl.multiple_of(gbase + go * GRP, 8)
        pltpu.sync_copy(srcr.at[pl.ds(row, GRP)], srcg.at[p])
        pltpu.sync_copy(dstr.at[pl.ds(row, GRP)], dstg.at[p])
        pltpu.sync_copy(wr.at[pl.ds(row, GRP)], wg.at[p])

    def _prefetch_group(go, p):
        row = pl.multiple_of(gbase + go * GRP, 8)
        pltpu.async_copy(srcr.at[pl.ds(row, GRP)], srcg.at[p], isem)
        pltpu.async_copy(dstr.at[pl.ds(row, GRP)], dstg.at[p], isem)
        pltpu.async_copy(wr.at[pl.ds(row, GRP)], wg.at[p], isem)

    def _wait_idx():
        for _ in range(3):
            pltpu.make_async_copy(
                srcr.at[pl.ds(0, GRP)], srcg.at[0], isem).wait()

    def _wait_one(sem):
        pltpu.make_async_copy(sup.at[pl.ds(0, CHUNK)], rows_v.at[0], sem).wait()

    def _scale_and_fire(p, g, slot):
        def scale_block(b, c2):
            wv = wg[p, g, pl.ds(b * 16, 16)]
            e0 = b * 16
            for l in range(16):
                ws = wv[l]
                for j in range(8):
                    sl = pl.ds(j * 16, 16)
                    rows_v[slot, e0 + l, sl] = rows_v[slot, e0 + l, sl] * ws
            return c2

        lax.fori_loop(0, CHUNK // 16, scale_block, 0)
        pltpu.async_copy(rows_v.at[slot], acc.at[dstg.at[p, g]], ssem,
                         add=True)

    _load_group_sync(0, 0)
    _prefetch_group(1, 1)

    def group_body(go, carry):
        p = lax.rem(go, 2)
        for g in range(GRP):
            c = go * GRP + g
            slot = g % NBUF

            @pl.when(c >= NBUF)
            def _():
                _wait_one(ssem)

            pltpu.async_copy(sup.at[srcg.at[p, g]], rows_v.at[slot], gsem)

            p_prev = p if g > 0 else 1 - p
            g_prev = (g - 1) % GRP
            slot_prev = (g - 1) % NBUF

            @pl.when(c >= 1)
            def _():
                _wait_one(gsem)
                _scale_and_fire(p_prev, g_prev, slot_prev)

            if g == 2:
                @pl.when(go < NGRP - 1)
                def _():
                    _prefetch_group(go + 1, 1 - p)
            if g == GRP - 1:
                @pl.when(go < NGRP - 1)
                def _():
                    _wait_idx()

        return carry

    lax.fori_loop(0, NGRP, group_body, 0)
    _wait_one(gsem)
    _scale_and_fire((NGRP - 1) % 2, GRP - 1, (GRP - 1) % NBUF)
    for _ in range(NBUF):
        _wait_one(ssem)
    plsc.subcore_barrier()

    def oloop(i, carry):
        r0 = row0 + i * ZCH
        pltpu.sync_copy(acc.at[pl.ds(r0, ZCH)], stage_v)
        pltpu.sync_copy(stage_v, out.at[cid, pl.ds(r0, ZCH)])
        return carry

    lax.fori_loop(0, ROWS_PER_SUB // ZCH, oloop, 0)


_spmm = pl.kernel(
    _spmm_body,
    out_type=jax.ShapeDtypeStruct((NC, N_PAD, D), jnp.float32),
    mesh=_mesh,
    scratch_types=[
        pltpu.VMEM_SHARED((N_PAD, D), jnp.float32),
        pltpu.VMEM((2, GRP, CHUNK), jnp.int32),
        pltpu.VMEM((2, GRP, CHUNK), jnp.int32),
        pltpu.VMEM((2, GRP, CHUNK), jnp.float32),
        pltpu.VMEM((NBUF, CHUNK, D), jnp.float32),
        pltpu.VMEM((ZCH, D), jnp.float32),
        pltpu.SemaphoreType.DMA,
        pltpu.SemaphoreType.DMA,
        pltpu.SemaphoreType.DMA,
    ],
)

_MBLK = 2000


def _mm1_body(x_ref, w_ref, b_ref, o_ref):
    o_ref[...] = (
        jnp.dot(x_ref[...], w_ref[...], preferred_element_type=jnp.float32)
        + b_ref[...]
    )


def _dense1(x, wt, b):
    return pl.pallas_call(
        _mm1_body,
        grid=(N_NODES // _MBLK,),
        in_specs=[
            pl.BlockSpec((_MBLK, D), lambda i: (i, 0)),
            pl.BlockSpec((D, D), lambda i: (0, 0)),
            pl.BlockSpec((1, D), lambda i: (0, 0)),
        ],
        out_specs=pl.BlockSpec((_MBLK, D), lambda i: (i, 0)),
        out_shape=jax.ShapeDtypeStruct((N_NODES, D), jnp.float32),
    )(x, wt, b.reshape(1, D))


def _mm2_body(hp_ref, w_ref, b_ref, o_ref):
    h = jax.nn.relu(hp_ref[0] + hp_ref[1])
    o_ref[...] = (
        jnp.dot(h, w_ref[...], preferred_element_type=jnp.float32)
        + b_ref[...]
    )


def _dense2(hp, wt, b):
    return pl.pallas_call(
        _mm2_body,
        grid=(N_NODES // _MBLK,),
        in_specs=[
            pl.BlockSpec((NC, _MBLK, D), lambda i: (0, i, 0)),
            pl.BlockSpec((D, D), lambda i: (0, 0)),
            pl.BlockSpec((1, D), lambda i: (0, 0)),
        ],
        out_specs=pl.BlockSpec((_MBLK, D), lambda i: (i, 0)),
        out_shape=jax.ShapeDtypeStruct((N_NODES, D), jnp.float32),
    )(hp, wt, b.reshape(1, D))


def _add_body(p_ref, o_ref):
    o_ref[...] = p_ref[0] + p_ref[1]


def _combine(p):
    return pl.pallas_call(
        _add_body,
        grid=(N_NODES // _MBLK,),
        in_specs=[pl.BlockSpec((NC, _MBLK, D), lambda i: (0, i, 0))],
        out_specs=pl.BlockSpec((_MBLK, D), lambda i: (i, 0)),
        out_shape=jax.ShapeDtypeStruct((N_NODES, D), jnp.float32),
    )(p)


def kernel(x, edge_index, edge_weight, W1, b1, W2, b2):
    src = edge_index[1].astype(jnp.int32)
    dst = edge_index[0].astype(jnp.int32)
    npad = E_PAD - src.shape[0]
    src_p = jnp.concatenate([src, jnp.zeros((npad,), jnp.int32)])
    dst_p = jnp.concatenate([dst, jnp.zeros((npad,), jnp.int32)])
    w_p = jnp.concatenate(
        [edge_weight.astype(jnp.float32), jnp.zeros((npad,), jnp.float32)]
    )
    src_p = src_p.reshape(E_PAD // CHUNK, CHUNK)
    dst_p = dst_p.reshape(E_PAD // CHUNK, CHUNK)
    w_p = w_p.reshape(E_PAD // CHUNK, CHUNK)

    s1 = _dense1(x, W1.T, b1)
    hp = _spmm(s1, src_p, dst_p, w_p)
    s2 = _dense2(hp, W2.T, b2)
    op = _spmm(s2, src_p, dst_p, w_p)
    return _combine(op)


# speedup vs baseline: 3.6661x; 1.0643x over previous
import jax
import jax.numpy as jnp
from jax import lax
from jax.experimental import pallas as pl
from jax.experimental.pallas import tpu as pltpu
from jax.experimental.pallas import tpu_sc as plsc

N_NODES = 10000
D = 128
HD = 64
N_EDGES = 320000
HN = 5000

NC = 2
NS = 16
NTILES = NC * NS
CHUNK = 128
E_PER_TILE = 10240
E_PAD = E_PER_TILE * NTILES
NCHUNKS = E_PER_TILE // CHUNK
R_PAD = 5120
ROWS_PER_SUB = R_PAD // NS
ZCH = 16
GRP = 8
NBUF = 2
NGRP = NCHUNKS // GRP

_mesh = plsc.VectorSubcoreMesh(core_axis_name="c", subcore_axis_name="s")


def _spmm_body(sup, srcr, dstr, wr, parr, out, table, acc, srcg, dstg, wg,
               parg, rows_v, stage_v, cstage, gsem, ssem, isem):
    cid = lax.axis_index("c")
    sid = lax.axis_index("s")
    wid = sid * NC + cid
    row0 = sid * ROWS_PER_SUB
    gbase = wid * NCHUNKS

    for i in range(ZCH):
        for j in range(D // 16):
            stage_v[i, pl.ds(j * 16, 16)] = jnp.zeros((16,), jnp.float32)

    def _load_group_sync(go, p):
        row = pl.multiple_of(gbase + go * GRP, 8)
        pltpu.sync_copy(srcr.at[pl.ds(row, GRP)], srcg.at[p])
        pltpu.sync_copy(dstr.at[pl.ds(row, GRP)], dstg.at[p])
        pltpu.sync_copy(wr.at[pl.ds(row, GRP)], wg.at[p])
        pltpu.sync_copy(parr.at[pl.ds(row, GRP)], parg.at[p])

    def _prefetch_group(go, p):
        row = pl.multiple_of(gbase + go * GRP, 8)
        pltpu.async_copy(srcr.at[pl.ds(row, GRP)], srcg.at[p], isem)
        pltpu.async_copy(dstr.at[pl.ds(row, GRP)], dstg.at[p], isem)
        pltpu.async_copy(wr.at[pl.ds(row, GRP)], wg.at[p], isem)
        pltpu.async_copy(parr.at[pl.ds(row, GRP)], parg.at[p], isem)

    def _wait_idx():
        for _ in range(4):
            pltpu.make_async_copy(
                srcr.at[pl.ds(0, GRP)], srcg.at[0], isem).wait()

    def _wait_one(sem):
        pltpu.make_async_copy(
            sup.at[pl.ds(0, CHUNK)], rows_v.at[0], sem).wait()

    def _scale_and_fire(p, g, slot):
        def scale_block(b, c2):
            wv = wg[p, g, pl.ds(b * 16, 16)]
            pv = parg[p, g, pl.ds(b * 16, 16)]
            e0 = b * 16
            zero16 = jnp.zeros((16,), jnp.float32)
            for l in range(16):
                ws = wv[l]
                pe = pv[l]
                ps = pl.multiple_of((pe & 1) * HD, 16)
                pd = pl.multiple_of(((pe >> 1) & 1) * HD, 16)
                e = e0 + l
                tmp = [
                    rows_v[slot, e, pl.ds(ps + j * 16, 16)] * ws
                    for j in range(HD // 16)
                ]
                for j in range(D // 16):
                    rows_v[slot, e, pl.ds(j * 16, 16)] = zero16
                for j in range(HD // 16):
                    rows_v[slot, e, pl.ds(pd + j * 16, 16)] = tmp[j]
            return c2

        lax.fori_loop(0, CHUNK // 16, scale_block, 0)
        pltpu.async_copy(rows_v.at[slot], acc.at[dstg.at[p, g]], ssem,
                         add=True)

    def group_body(go, carry):
        p = lax.rem(go, 2)
        for g in range(GRP):
            c = go * GRP + g
            slot = g % NBUF

            @pl.when(c >= NBUF)
            def _():
                _wait_one(ssem)

            pltpu.async_copy(table.at[srcg.at[p, g]], rows_v.at[slot], gsem)

            p_prev = p if g > 0 else 1 - p
            g_prev = (g - 1) % GRP
            slot_prev = (g - 1) % NBUF

            @pl.when(c >= 1)
            def _():
                _wait_one(gsem)
                _scale_and_fire(p_prev, g_prev, slot_prev)

            if g == 2:
                @pl.when(go < NGRP - 1)
                def _():
                    _prefetch_group(go + 1, 1 - p)
            if g == GRP - 1:
                @pl.when(go < NGRP - 1)
                def _():
                    _wait_idx()

        return carry

    def half_body(h, carry):
        def zloop(i, c2):
            pltpu.sync_copy(stage_v, acc.at[pl.ds(row0 + i * ZCH, ZCH)])
            return c2

        lax.fori_loop(0, ROWS_PER_SUB // ZCH, zloop, 0)
        srow = pl.multiple_of(h * R_PAD + row0, 8)
        pltpu.sync_copy(sup.at[pl.ds(srow, ROWS_PER_SUB)],
                        table.at[pl.ds(row0, ROWS_PER_SUB)])
        _load_group_sync(0, 0)
        _prefetch_group(1, 1)
        plsc.subcore_barrier()

        lax.fori_loop(0, NGRP, group_body, 0)
        _wait_one(gsem)
        _scale_and_fire((NGRP - 1) % 2, GRP - 1, (GRP - 1) % NBUF)
        for _ in range(NBUF):
            _wait_one(ssem)
        plsc.subcore_barrier()

        obase = (cid * 2 + h) * R_PAD

        def oloop(i, c2):
            r0 = row0 + i * ZCH
            pltpu.sync_copy(acc.at[pl.ds(r0, ZCH)], cstage)
            orow = pl.multiple_of(obase + r0, 8)
            pltpu.sync_copy(cstage, out.at[pl.ds(orow, ZCH)])
            return c2

        lax.fori_loop(0, ROWS_PER_SUB // ZCH, oloop, 0)
        return carry

    lax.fori_loop(0, 2, half_body, 0)


_spmm = pl.kernel(
    _spmm_body,
    out_type=jax.ShapeDtypeStruct((NC * 2 * R_PAD, D), jnp.float32),
    mesh=_mesh,
    scratch_types=[
        pltpu.VMEM_SHARED((R_PAD, D), jnp.float32),
        pltpu.VMEM_SHARED((R_PAD, D), jnp.float32),
        pltpu.VMEM((2, GRP, CHUNK), jnp.int32),
        pltpu.VMEM((2, GRP, CHUNK), jnp.int32),
        pltpu.VMEM((2, GRP, CHUNK), jnp.float32),
        pltpu.VMEM((2, GRP, CHUNK), jnp.int32),
        pltpu.VMEM((NBUF, CHUNK, D), jnp.float32),
        pltpu.VMEM((ZCH, D), jnp.float32),
        pltpu.VMEM((ZCH, D), jnp.float32),
        pltpu.SemaphoreType.DMA,
        pltpu.SemaphoreType.DMA,
        pltpu.SemaphoreType.DMA,
    ],
)

_MBLK = 1000


def _mm1_body(xa_ref, xb_ref, w_ref, b_ref, o_ref):
    h = pl.program_id(1)
    sa = jnp.dot(xa_ref[...], w_ref[...], preferred_element_type=jnp.float32)
    sb = jnp.dot(xb_ref[...], w_ref[...], preferred_element_type=jnp.float32)
    sa = sa + b_ref[...]
    sb = sb + b_ref[...]
    sa_h = jnp.where(h == 0, sa[:, :HD], sa[:, HD:])
    sb_h = jnp.where(h == 0, sb[:, :HD], sb[:, HD:])
    o_ref[0] = jnp.concatenate([sa_h, sb_h], axis=1)


def _dense1(x, wt, b):
    return pl.pallas_call(
        _mm1_body,
        grid=(HN // _MBLK, 2),
        in_specs=[
            pl.BlockSpec((_MBLK, D), lambda i, h: (i, 0)),
            pl.BlockSpec((_MBLK, D), lambda i, h: (i + HN // _MBLK, 0)),
            pl.BlockSpec((D, D), lambda i, h: (0, 0)),
            pl.BlockSpec((1, D), lambda i, h: (0, 0)),
        ],
        out_specs=pl.BlockSpec((1, _MBLK, D), lambda i, h: (h, i, 0)),
        out_shape=jax.ShapeDtypeStruct((2, R_PAD, D), jnp.float32),
    )(x, x, wt, b.reshape(1, D))


def _mm2_body(hp_ref, w_ref, b_ref, o_ref):
    h = pl.program_id(1)
    ha = jax.nn.relu(jnp.concatenate(
        [hp_ref[0, 0, :, :HD] + hp_ref[1, 0, :, :HD],
         hp_ref[0, 1, :, :HD] + hp_ref[1, 1, :, :HD]], axis=1))
    hb = jax.nn.relu(jnp.concatenate(
        [hp_ref[0, 0, :, HD:] + hp_ref[1, 0, :, HD:],
         hp_ref[0, 1, :, HD:] + hp_ref[1, 1, :, HD:]], axis=1))
    sa = jnp.dot(ha, w_ref[...], preferred_element_type=jnp.float32)
    sb = jnp.dot(hb, w_ref[...], preferred_element_type=jnp.float32)
    sa = sa + b_ref[...]
    sb = sb + b_ref[...]
    sa_h = jnp.where(h == 0, sa[:, :HD], sa[:, HD:])
    sb_h = jnp.where(h == 0, sb[:, :HD], sb[:, HD:])
    o_ref[0] = jnp.concatenate([sa_h, sb_h], axis=1)


def _dense2(hp, wt, b):
    return pl.pallas_call(
        _mm2_body,
        grid=(HN // _MBLK, 2),
        in_specs=[
            pl.BlockSpec((NC, 2, _MBLK, D), lambda i, h: (0, 0, i, 0)),
            pl.BlockSpec((D, D), lambda i, h: (0, 0)),
            pl.BlockSpec((1, D), lambda i, h: (0, 0)),
        ],
        out_specs=pl.BlockSpec((1, _MBLK, D), lambda i, h: (h, i, 0)),
        out_shape=jax.ShapeDtypeStruct((2, R_PAD, D), jnp.float32),
    )(hp, wt, b.reshape(1, D))


def _add_body(p_ref, o_ref):
    i = pl.program_id(0)
    sel = i // (HN // _MBLK)
    cols0 = jnp.where(
        sel == 0, p_ref[0, 0, :, :HD] + p_ref[1, 0, :, :HD],
        p_ref[0, 0, :, HD:] + p_ref[1, 0, :, HD:])
    cols1 = jnp.where(
        sel == 0, p_ref[0, 1, :, :HD] + p_ref[1, 1, :, :HD],
        p_ref[0, 1, :, HD:] + p_ref[1, 1, :, HD:])
    o_ref[...] = jnp.concatenate([cols0, cols1], axis=1)


def _combine(p):
    nb = HN // _MBLK
    return pl.pallas_call(
        _add_body,
        grid=(2 * nb,),
        in_specs=[
            pl.BlockSpec((NC, 2, _MBLK, D), lambda i: (0, 0, i % nb, 0))
        ],
        out_specs=pl.BlockSpec((_MBLK, D), lambda i: (i, 0)),
        out_shape=jax.ShapeDtypeStruct((N_NODES, D), jnp.float32),
    )(p)


def kernel(x, edge_index, edge_weight, W1, b1, W2, b2):
    src = edge_index[1].astype(jnp.int32)
    dst = edge_index[0].astype(jnp.int32)
    npad = E_PAD - src.shape[0]
    src = jnp.concatenate([src, jnp.zeros((npad,), jnp.int32)])
    dst = jnp.concatenate([dst, jnp.zeros((npad,), jnp.int32)])
    w_p = jnp.concatenate(
        [edge_weight.astype(jnp.float32), jnp.zeros((npad,), jnp.float32)]
    )
    srow = jnp.where(src >= HN, src - HN, src)
    drow = jnp.where(dst >= HN, dst - HN, dst)
    par = (src >= HN).astype(jnp.int32) + 2 * (dst >= HN).astype(jnp.int32)
    srow = srow.reshape(E_PAD // CHUNK, CHUNK)
    drow = drow.reshape(E_PAD // CHUNK, CHUNK)
    w_p = w_p.reshape(E_PAD // CHUNK, CHUNK)
    par = par.reshape(E_PAD // CHUNK, CHUNK)

    s1 = _dense1(x, W1.T, b1)
    hp = _spmm(s1.reshape(2 * R_PAD, D), srow, drow, w_p, par)
    s2 = _dense2(hp.reshape(NC, 2, R_PAD, D), W2.T, b2)
    op = _spmm(s2.reshape(2 * R_PAD, D), srow, drow, w_p, par)
    return _combine(op.reshape(NC, 2, R_PAD, D))


# speedup vs baseline: 4.7373x; 1.2922x over previous
import jax
import jax.numpy as jnp
from jax import lax
from jax.experimental import pallas as pl
from jax.experimental.pallas import tpu as pltpu
from jax.experimental.pallas import tpu_sc as plsc

N_NODES = 10000
D = 128
HD = 64
N_EDGES = 320000
HN = 5000

NC = 2
NS = 16
NTILES = NC * NS
CHUNK = 128
E_PER_TILE = 10240
E_PAD = E_PER_TILE * NTILES
NCHUNKS = E_PER_TILE // CHUNK
R_PAD = 5120
ROWS_PER_SUB = R_PAD // NS
ZCH = 16
GRP = 8
NBUF = 2
NGRP = NCHUNKS // GRP

_mesh = plsc.VectorSubcoreMesh(core_axis_name="c", subcore_axis_name="s")


def _spmm_body(sup, srcr, dstr, wr, parr, out, table, acc, srcg, dstg, wg,
               parg, rows_v, stage_v, cstage, gsem, ssem, isem):
    cid = lax.axis_index("c")
    sid = lax.axis_index("s")
    wid = sid * NC + cid
    row0 = sid * ROWS_PER_SUB
    gbase = wid * NCHUNKS

    for i in range(ZCH):
        for j in range(D // 16):
            stage_v[i, pl.ds(j * 16, 16)] = jnp.zeros((16,), jnp.float32)

    def _load_group_sync(go, p):
        row = pl.multiple_of(gbase + go * GRP, 8)
        pltpu.sync_copy(srcr.at[pl.ds(row, GRP)], srcg.at[p])
        pltpu.sync_copy(dstr.at[pl.ds(row, GRP)], dstg.at[p])
        pltpu.sync_copy(wr.at[pl.ds(row, GRP)], wg.at[p])
        pltpu.sync_copy(parr.at[pl.ds(row, GRP)], parg.at[p])

    def _prefetch_group(go, p):
        row = pl.multiple_of(gbase + go * GRP, 8)
        pltpu.async_copy(srcr.at[pl.ds(row, GRP)], srcg.at[p], isem)
        pltpu.async_copy(dstr.at[pl.ds(row, GRP)], dstg.at[p], isem)
        pltpu.async_copy(wr.at[pl.ds(row, GRP)], wg.at[p], isem)
        pltpu.async_copy(parr.at[pl.ds(row, GRP)], parg.at[p], isem)

    def _wait_idx():
        for _ in range(4):
            pltpu.make_async_copy(
                srcr.at[pl.ds(0, GRP)], srcg.at[0], isem).wait()

    def _wait_one(sem):
        pltpu.make_async_copy(
            sup.at[pl.ds(0, CHUNK)], rows_v.at[0], sem).wait()

    def _scale_and_fire(p, g, slot):
        def scale_block(b, c2):
            wv = wg[p, g, pl.ds(b * 16, 16)]
            pv = parg[p, g, pl.ds(b * 16, 16)]
            e0 = b * 16
            zero16 = jnp.zeros((16,), jnp.float32)
            for l in range(16):
                ws = wv[l]
                pe = pv[l]
                pdb = (pe >> 1) & 1
                ps = pl.multiple_of((pe & 1) * HD, 16)
                pd = pl.multiple_of(pdb * HD, 16)
                pz = pl.multiple_of((1 - pdb) * HD, 16)
                e = e0 + l
                tmp = [
                    rows_v[slot, e, pl.ds(ps + j * 16, 16)] * ws
                    for j in range(HD // 16)
                ]
                for j in range(HD // 16):
                    rows_v[slot, e, pl.ds(pz + j * 16, 16)] = zero16
                for j in range(HD // 16):
                    rows_v[slot, e, pl.ds(pd + j * 16, 16)] = tmp[j]
            return c2

        lax.fori_loop(0, CHUNK // 16, scale_block, 0)
        pltpu.async_copy(rows_v.at[slot], acc.at[dstg.at[p, g]], ssem,
                         add=True)

    def group_body(go, carry):
        p = lax.rem(go, 2)
        for g in range(GRP):
            c = go * GRP + g
            slot = g % NBUF

            @pl.when(c >= NBUF)
            def _():
                _wait_one(ssem)

            pltpu.async_copy(table.at[srcg.at[p, g]], rows_v.at[slot], gsem)

            p_prev = p if g > 0 else 1 - p
            g_prev = (g - 1) % GRP
            slot_prev = (g - 1) % NBUF

            @pl.when(c >= 1)
            def _():
                _wait_one(gsem)
                _scale_and_fire(p_prev, g_prev, slot_prev)

            if g == 2:
                @pl.when(go < NGRP - 1)
                def _():
                    _prefetch_group(go + 1, 1 - p)
            if g == GRP - 1:
                @pl.when(go < NGRP - 1)
                def _():
                    _wait_idx()

        return carry

    def half_body(h, carry):
        def zloop(i, c2):
            pltpu.sync_copy(stage_v, acc.at[pl.ds(row0 + i * ZCH, ZCH)])
            return c2

        lax.fori_loop(0, ROWS_PER_SUB // ZCH, zloop, 0)
        srow = pl.multiple_of(h * R_PAD + row0, 8)
        pltpu.sync_copy(sup.at[pl.ds(srow, ROWS_PER_SUB)],
                        table.at[pl.ds(row0, ROWS_PER_SUB)])
        _load_group_sync(0, 0)
        _prefetch_group(1, 1)
        plsc.subcore_barrier()

        lax.fori_loop(0, NGRP, group_body, 0)
        _wait_one(gsem)
        _scale_and_fire((NGRP - 1) % 2, GRP - 1, (GRP - 1) % NBUF)
        for _ in range(NBUF):
            _wait_one(ssem)
        plsc.subcore_barrier()

        obase = (cid * 2 + h) * R_PAD

        def oloop(i, c2):
            r0 = row0 + i * ZCH
            pltpu.sync_copy(acc.at[pl.ds(r0, ZCH)], cstage)
            orow = pl.multiple_of(obase + r0, 8)
            pltpu.sync_copy(cstage, out.at[pl.ds(orow, ZCH)])
            return c2

        lax.fori_loop(0, ROWS_PER_SUB // ZCH, oloop, 0)
        return carry

    lax.fori_loop(0, 2, half_body, 0)


_spmm = pl.kernel(
    _spmm_body,
    out_type=jax.ShapeDtypeStruct((NC * 2 * R_PAD, D), jnp.float32),
    mesh=_mesh,
    scratch_types=[
        pltpu.VMEM_SHARED((R_PAD, D), jnp.float32),
        pltpu.VMEM_SHARED((R_PAD, D), jnp.float32),
        pltpu.VMEM((2, GRP, CHUNK), jnp.int32),
        pltpu.VMEM((2, GRP, CHUNK), jnp.int32),
        pltpu.VMEM((2, GRP, CHUNK), jnp.float32),
        pltpu.VMEM((2, GRP, CHUNK), jnp.int32),
        pltpu.VMEM((NBUF, CHUNK, D), jnp.float32),
        pltpu.VMEM((ZCH, D), jnp.float32),
        pltpu.VMEM((ZCH, D), jnp.float32),
        pltpu.SemaphoreType.DMA,
        pltpu.SemaphoreType.DMA,
        pltpu.SemaphoreType.DMA,
    ],
)

_MBLK = 1000


def _mm1_body(xa_ref, xb_ref, w_ref, b_ref, o_ref):
    h = pl.program_id(1)
    sa = jnp.dot(xa_ref[...], w_ref[...], preferred_element_type=jnp.float32)
    sb = jnp.dot(xb_ref[...], w_ref[...], preferred_element_type=jnp.float32)
    sa = sa + b_ref[...]
    sb = sb + b_ref[...]
    sa_h = jnp.where(h == 0, sa[:, :HD], sa[:, HD:])
    sb_h = jnp.where(h == 0, sb[:, :HD], sb[:, HD:])
    o_ref[0] = jnp.concatenate([sa_h, sb_h], axis=1)


def _dense1(x, wt, b):
    return pl.pallas_call(
        _mm1_body,
        grid=(HN // _MBLK, 2),
        in_specs=[
            pl.BlockSpec((_MBLK, D), lambda i, h: (i, 0)),
            pl.BlockSpec((_MBLK, D), lambda i, h: (i + HN // _MBLK, 0)),
            pl.BlockSpec((D, D), lambda i, h: (0, 0)),
            pl.BlockSpec((1, D), lambda i, h: (0, 0)),
        ],
        out_specs=pl.BlockSpec((1, _MBLK, D), lambda i, h: (h, i, 0)),
        out_shape=jax.ShapeDtypeStruct((2, R_PAD, D), jnp.float32),
    )(x, x, wt, b.reshape(1, D))


def _mm2_body(hp_ref, w_ref, b_ref, o_ref):
    h = pl.program_id(1)
    ha = jax.nn.relu(jnp.concatenate(
        [hp_ref[0, 0, :, :HD] + hp_ref[1, 0, :, :HD],
         hp_ref[0, 1, :, :HD] + hp_ref[1, 1, :, :HD]], axis=1))
    hb = jax.nn.relu(jnp.concatenate(
        [hp_ref[0, 0, :, HD:] + hp_ref[1, 0, :, HD:],
         hp_ref[0, 1, :, HD:] + hp_ref[1, 1, :, HD:]], axis=1))
    sa = jnp.dot(ha, w_ref[...], preferred_element_type=jnp.float32)
    sb = jnp.dot(hb, w_ref[...], preferred_element_type=jnp.float32)
    sa = sa + b_ref[...]
    sb = sb + b_ref[...]
    sa_h = jnp.where(h == 0, sa[:, :HD], sa[:, HD:])
    sb_h = jnp.where(h == 0, sb[:, :HD], sb[:, HD:])
    o_ref[0] = jnp.concatenate([sa_h, sb_h], axis=1)


def _dense2(hp, wt, b):
    return pl.pallas_call(
        _mm2_body,
        grid=(HN // _MBLK, 2),
        in_specs=[
            pl.BlockSpec((NC, 2, _MBLK, D), lambda i, h: (0, 0, i, 0)),
            pl.BlockSpec((D, D), lambda i, h: (0, 0)),
            pl.BlockSpec((1, D), lambda i, h: (0, 0)),
        ],
        out_specs=pl.BlockSpec((1, _MBLK, D), lambda i, h: (h, i, 0)),
        out_shape=jax.ShapeDtypeStruct((2, R_PAD, D), jnp.float32),
    )(hp, wt, b.reshape(1, D))


def _add_body(p_ref, o_ref):
    i = pl.program_id(0)
    sel = i // (HN // _MBLK)
    cols0 = jnp.where(
        sel == 0, p_ref[0, 0, :, :HD] + p_ref[1, 0, :, :HD],
        p_ref[0, 0, :, HD:] + p_ref[1, 0, :, HD:])
    cols1 = jnp.where(
        sel == 0, p_ref[0, 1, :, :HD] + p_ref[1, 1, :, :HD],
        p_ref[0, 1, :, HD:] + p_ref[1, 1, :, HD:])
    o_ref[...] = jnp.concatenate([cols0, cols1], axis=1)


def _combine(p):
    nb = HN // _MBLK
    return pl.pallas_call(
        _add_body,
        grid=(2 * nb,),
        in_specs=[
            pl.BlockSpec((NC, 2, _MBLK, D), lambda i: (0, 0, i % nb, 0))
        ],
        out_specs=pl.BlockSpec((_MBLK, D), lambda i: (i, 0)),
        out_shape=jax.ShapeDtypeStruct((N_NODES, D), jnp.float32),
    )(p)


def kernel(x, edge_index, edge_weight, W1, b1, W2, b2):
    src = edge_index[1].astype(jnp.int32)
    dst = edge_index[0].astype(jnp.int32)
    npad = E_PAD - src.shape[0]
    src = jnp.concatenate([src, jnp.zeros((npad,), jnp.int32)])
    dst = jnp.concatenate([dst, jnp.zeros((npad,), jnp.int32)])
    w_p = jnp.concatenate(
        [edge_weight.astype(jnp.float32), jnp.zeros((npad,), jnp.float32)]
    )
    srow = jnp.where(src >= HN, src - HN, src)
    drow = jnp.where(dst >= HN, dst - HN, dst)
    par = (src >= HN).astype(jnp.int32) + 2 * (dst >= HN).astype(jnp.int32)
    srow = srow.reshape(E_PAD // CHUNK, CHUNK)
    drow = drow.reshape(E_PAD // CHUNK, CHUNK)
    w_p = w_p.reshape(E_PAD // CHUNK, CHUNK)
    par = par.reshape(E_PAD // CHUNK, CHUNK)

    s1 = _dense1(x, W1.T, b1)
    hp = _spmm(s1.reshape(2 * R_PAD, D), srow, drow, w_p, par)
    s2 = _dense2(hp.reshape(NC, 2, R_PAD, D), W2.T, b2)
    op = _spmm(s2.reshape(2 * R_PAD, D), srow, drow, w_p, par)
    return _combine(op.reshape(NC, 2, R_PAD, D))
